# baseline (speedup 1.0000x reference)
import functools

import jax
import jax.numpy as jnp
from jax import lax
from jax.experimental import pallas as pl
from jax.experimental.pallas import tpu as pltpu
from jax.experimental.pallas import tpu_sc as plsc

_B, _C, _H, _W, _N = 16, 96, 56, 56, 16384
_CP = 128
_IM = 224
_NC, _NS, _L = 2, 16, 16
_NW = _NC * _NS
_PTS = _B * _N
_PT = _PTS // _NW
_CHUNK = 32
_NCH = _PT // _CHUNK
_NPAIR = _NCH // 2


def _compute_taps(ux_v, uy_v, idx_ref, w_ref, off, brow):
    for g in range(_CHUNK // _L):
        xu = ux_v[pl.ds(off + g * _L, _L)]
        yv = uy_v[pl.ds(off + g * _L, _L)]
        gx = (xu * 2.0) / float(_IM) + (-1.0)
        gy = (yv * 2.0) / float(_IM) + (-1.0)
        x = ((gx + 1.0) * float(_W) - 1.0) / 2.0
        y = ((gy + 1.0) * float(_H) - 1.0) / 2.0
        xi = (x + 8.0).astype(jnp.int32) - 8
        xf = xi.astype(jnp.float32)
        xover = xf > x
        xi = jnp.where(xover, xi - 1, xi)
        xf = jnp.where(xover, xf - 1.0, xf)
        yi = (y + 8.0).astype(jnp.int32) - 8
        yf = yi.astype(jnp.float32)
        yover = yf > y
        yi = jnp.where(yover, yi - 1, yi)
        yf = jnp.where(yover, yf - 1.0, yf)

        wx1 = x - xf
        wx0 = 1.0 - wx1
        wy1 = y - yf
        wy0 = 1.0 - wy1
        zero = jnp.zeros((_L,), jnp.float32)
        wx0 = jnp.where(xi >= 0, wx0, zero)
        wx1 = jnp.where(xi <= _W - 2, wx1, zero)
        wy0 = jnp.where(yi >= 0, wy0, zero)
        wy1 = jnp.where(yi <= _H - 2, wy1, zero)

        xc0 = jnp.clip(xi, 0, _W - 1)
        xc1 = jnp.clip(xi + 1, 0, _W - 1)
        yc0 = jnp.clip(yi, 0, _H - 1)
        yc1 = jnp.clip(yi + 1, 0, _H - 1)
        r0 = brow + yc0 * _W
        r1 = brow + yc1 * _W

        s = g * _L
        idx_ref[pl.ds(0 * _CHUNK + s, _L)] = r0 + xc0
        idx_ref[pl.ds(1 * _CHUNK + s, _L)] = r0 + xc1
        idx_ref[pl.ds(2 * _CHUNK + s, _L)] = r1 + xc0
        idx_ref[pl.ds(3 * _CHUNK + s, _L)] = r1 + xc1
        w_ref[pl.ds(0 * _CHUNK + s, _L)] = wy0 * wx0
        w_ref[pl.ds(1 * _CHUNK + s, _L)] = wy0 * wx1
        w_ref[pl.ds(2 * _CHUNK + s, _L)] = wy1 * wx0
        w_ref[pl.ds(3 * _CHUNK + s, _L)] = wy1 * wx1


def _combine(g_ref, w_ref, o_ref):
    lane = lax.iota(jnp.int32, _L)
    pat0 = (lane & 3) * _CHUNK + lax.shift_right_logical(lane, 2)
    for q in range(_CHUNK // 4):
        wq = plsc.load_gather(w_ref, [pat0 + (4 * q)])
        for i in range(4):
            p = q * 4 + i
            w00, w01, w10, w11 = (wq[4 * i], wq[4 * i + 1],
                                  wq[4 * i + 2], wq[4 * i + 3])
            for j in range(_C // _L):
                cs = pl.ds(j * _L, _L)
                acc = w00 * g_ref[0 * _CHUNK + p, cs]
                acc = acc + w01 * g_ref[1 * _CHUNK + p, cs]
                acc = acc + w10 * g_ref[2 * _CHUNK + p, cs]
                acc = acc + w11 * g_ref[3 * _CHUNK + p, cs]
                o_ref[p >> 3, p & 7, cs] = acc


def _sc_interp(ft, ux, uy):
    mesh = plsc.VectorSubcoreMesh(core_axis_name="c", subcore_axis_name="s")

    @functools.partial(
        pl.kernel,
        mesh=mesh,
        compiler_params=pltpu.CompilerParams(use_tc_tiling_on_sc=False, needs_layout_passes=False),
        out_type=jax.ShapeDtypeStruct((_PTS // 8, 8, _C), jnp.float32),
        scratch_types=[
            pltpu.VMEM((_PT,), jnp.float32),
            pltpu.VMEM((_PT,), jnp.float32),
            pltpu.VMEM((4 * _CHUNK,), jnp.int32),
            pltpu.VMEM((4 * _CHUNK,), jnp.int32),
            pltpu.VMEM((4 * _CHUNK,), jnp.float32),
            pltpu.VMEM((4 * _CHUNK,), jnp.float32),
            pltpu.VMEM((4 * _CHUNK, _C), jnp.float32),
            pltpu.VMEM((4 * _CHUNK, _C), jnp.float32),
            pltpu.VMEM((_CHUNK // 8, 8, _C), jnp.float32),
            pltpu.VMEM((_CHUNK // 8, 8, _C), jnp.float32),
            pltpu.SemaphoreType.DMA,
            pltpu.SemaphoreType.DMA,
            pltpu.SemaphoreType.DMA,
            pltpu.SemaphoreType.DMA,
        ],
    )
    def body(ft_hbm, ux_hbm, uy_hbm, out_hbm,
             ux_v, uy_v, ia_v, ib_v, wa_v, wb_v, ga_v, gb_v, oa_v, ob_v,
             sem_a, sem_b, osem_a, osem_b):
        wid = lax.axis_index("s") * _NC + lax.axis_index("c")
        base_pt = wid * _PT
        brow = (wid // 2) * (_H * _W)
        pltpu.sync_copy(ux_hbm.at[pl.ds(base_pt, _PT)], ux_v)
        pltpu.sync_copy(uy_hbm.at[pl.ds(base_pt, _PT)], uy_v)

        def out_slice(off):
            return out_hbm.at[pl.ds((base_pt + off) // 8, _CHUNK // 8)]

        _compute_taps(ux_v, uy_v, ia_v, wa_v, 0, brow)
        pltpu.async_copy(ft_hbm.at[ia_v], ga_v, sem_a)

        def pair(k, _):
            off0 = (2 * k) * _CHUNK
            off1 = off0 + _CHUNK
            _compute_taps(ux_v, uy_v, ib_v, wb_v, off1, brow)
            pltpu.async_copy(ft_hbm.at[ib_v], gb_v, sem_b)
            pltpu.make_async_copy(ft_hbm.at[ia_v], ga_v, sem_a).wait()

            @pl.when(k > 0)
            def _():
                pltpu.make_async_copy(oa_v, out_slice(off0 - 2 * _CHUNK), osem_a).wait()

            _combine(ga_v, wa_v, oa_v)
            pltpu.async_copy(oa_v, out_slice(off0), osem_a)

            @pl.when(k < _NPAIR - 1)
            def _():
                _compute_taps(ux_v, uy_v, ia_v, wa_v, off1 + _CHUNK, brow)
                pltpu.async_copy(ft_hbm.at[ia_v], ga_v, sem_a)

            pltpu.make_async_copy(ft_hbm.at[ib_v], gb_v, sem_b).wait()

            @pl.when(k > 0)
            def _():
                pltpu.make_async_copy(ob_v, out_slice(off1 - 2 * _CHUNK), osem_b).wait()

            _combine(gb_v, wb_v, ob_v)
            pltpu.async_copy(ob_v, out_slice(off1), osem_b)
            return 0

        lax.fori_loop(0, _NPAIR, pair, 0)

        pltpu.make_async_copy(oa_v, out_slice((_NCH - 2) * _CHUNK), osem_a).wait()
        pltpu.make_async_copy(ob_v, out_slice((_NCH - 1) * _CHUNK), osem_b).wait()

    return body(ft, ux, uy)


def kernel(f, uv):
    ft = jnp.transpose(f, (0, 2, 3, 1)).reshape(_B * _H * _W, _C)
    ux = uv[:, :, 1].reshape(-1)
    uy = uv[:, :, 0].reshape(-1)
    out = _sc_interp(ft, ux, uy)
    return out.reshape(_B, _N, _C)


# speedup vs baseline: 1.1974x; 1.1974x over previous
import functools

import jax
import jax.numpy as jnp
from jax import lax
from jax.experimental import pallas as pl
from jax.experimental.pallas import tpu as pltpu
from jax.experimental.pallas import tpu_sc as plsc

_B, _C, _H, _W, _N = 16, 96, 56, 56, 16384
_CP = 128
_IM = 224
_NC, _NS, _L = 2, 16, 16
_NW = _NC * _NS
_PTS = _B * _N
_PT = _PTS // _NW
_CHUNK = 32
_NCH = _PT // _CHUNK
_NPAIR = _NCH // 2


def _compute_taps(ux_v, uy_v, idx_ref, w_ref, off, brow):
    for g in range(_CHUNK // _L):
        xu = ux_v[pl.ds(off + g * _L, _L)]
        yv = uy_v[pl.ds(off + g * _L, _L)]
        gx = (xu * 2.0) / float(_IM) + (-1.0)
        gy = (yv * 2.0) / float(_IM) + (-1.0)
        x = ((gx + 1.0) * float(_W) - 1.0) / 2.0
        y = ((gy + 1.0) * float(_H) - 1.0) / 2.0
        xi = (x + 8.0).astype(jnp.int32) - 8
        xf = xi.astype(jnp.float32)
        xover = xf > x
        xi = jnp.where(xover, xi - 1, xi)
        xf = jnp.where(xover, xf - 1.0, xf)
        yi = (y + 8.0).astype(jnp.int32) - 8
        yf = yi.astype(jnp.float32)
        yover = yf > y
        yi = jnp.where(yover, yi - 1, yi)
        yf = jnp.where(yover, yf - 1.0, yf)

        wx1 = x - xf
        wx0 = 1.0 - wx1
        wy1 = y - yf
        wy0 = 1.0 - wy1
        zero = jnp.zeros((_L,), jnp.float32)
        wx0 = jnp.where(xi >= 0, wx0, zero)
        wx1 = jnp.where(xi <= _W - 2, wx1, zero)
        wy0 = jnp.where(yi >= 0, wy0, zero)
        wy1 = jnp.where(yi <= _H - 2, wy1, zero)

        xc0 = jnp.clip(xi, 0, _W - 1)
        xc1 = jnp.clip(xi + 1, 0, _W - 1)
        yc0 = jnp.clip(yi, 0, _H - 1)
        yc1 = jnp.clip(yi + 1, 0, _H - 1)
        r0 = brow + yc0 * _W
        r1 = brow + yc1 * _W

        s = g * _L
        idx_ref[pl.ds(0 * _CHUNK + s, _L)] = r0 + xc0
        idx_ref[pl.ds(1 * _CHUNK + s, _L)] = r0 + xc1
        idx_ref[pl.ds(2 * _CHUNK + s, _L)] = r1 + xc0
        idx_ref[pl.ds(3 * _CHUNK + s, _L)] = r1 + xc1
        w_ref[pl.ds(0 * _CHUNK + s, _L)] = wy0 * wx0
        w_ref[pl.ds(1 * _CHUNK + s, _L)] = wy0 * wx1
        w_ref[pl.ds(2 * _CHUNK + s, _L)] = wy1 * wx0
        w_ref[pl.ds(3 * _CHUNK + s, _L)] = wy1 * wx1


def _combine(g_ref, w_ref, o_ref):
    lane = lax.iota(jnp.int32, _L)
    pat0 = (lane & 3) * _CHUNK + lax.shift_right_logical(lane, 2)
    for q in range(_CHUNK // 4):
        wq = plsc.load_gather(w_ref, [pat0 + (4 * q)])
        for i in range(4):
            p = q * 4 + i
            w00, w01, w10, w11 = (wq[4 * i], wq[4 * i + 1],
                                  wq[4 * i + 2], wq[4 * i + 3])
            for j in range(_C // _L):
                cs = pl.ds(j * _L, _L)
                acc = w00 * g_ref[0 * _CHUNK + p, cs]
                acc = acc + w01 * g_ref[1 * _CHUNK + p, cs]
                acc = acc + w10 * g_ref[2 * _CHUNK + p, cs]
                acc = acc + w11 * g_ref[3 * _CHUNK + p, cs]
                o_ref[p >> 3, p & 7, cs] = acc


def _sc_interp(ft, ux, uy):
    mesh = plsc.VectorSubcoreMesh(core_axis_name="c", subcore_axis_name="s")

    @functools.partial(
        pl.kernel,
        mesh=mesh,
        compiler_params=pltpu.CompilerParams(needs_layout_passes=False),
        out_type=jax.ShapeDtypeStruct((_PTS // 8, 8, _C), jnp.float32),
        scratch_types=[
            pltpu.VMEM((_PT,), jnp.float32),
            pltpu.VMEM((_PT,), jnp.float32),
            pltpu.VMEM((4 * _CHUNK,), jnp.int32),
            pltpu.VMEM((4 * _CHUNK,), jnp.int32),
            pltpu.VMEM((4 * _CHUNK,), jnp.float32),
            pltpu.VMEM((4 * _CHUNK,), jnp.float32),
            pltpu.VMEM((4 * _CHUNK, _CP), jnp.float32),
            pltpu.VMEM((4 * _CHUNK, _CP), jnp.float32),
            pltpu.VMEM((_CHUNK // 8, 8, _C), jnp.float32),
            pltpu.VMEM((_CHUNK // 8, 8, _C), jnp.float32),
            pltpu.SemaphoreType.DMA,
            pltpu.SemaphoreType.DMA,
            pltpu.SemaphoreType.DMA,
            pltpu.SemaphoreType.DMA,
        ],
    )
    def body(ft_hbm, ux_hbm, uy_hbm, out_hbm,
             ux_v, uy_v, ia_v, ib_v, wa_v, wb_v, ga_v, gb_v, oa_v, ob_v,
             sem_a, sem_b, osem_a, osem_b):
        wid = lax.axis_index("s") * _NC + lax.axis_index("c")
        base_pt = wid * _PT
        brow = (wid // 2) * (_H * _W)
        pltpu.sync_copy(ux_hbm.at[pl.ds(base_pt, _PT)], ux_v)
        pltpu.sync_copy(uy_hbm.at[pl.ds(base_pt, _PT)], uy_v)

        def out_slice(off):
            return out_hbm.at[pl.ds((base_pt + off) // 8, _CHUNK // 8)]

        _compute_taps(ux_v, uy_v, ia_v, wa_v, 0, brow)
        pltpu.async_copy(ft_hbm.at[ia_v], ga_v, sem_a)

        def pair(k, _):
            off0 = (2 * k) * _CHUNK
            off1 = off0 + _CHUNK
            _compute_taps(ux_v, uy_v, ib_v, wb_v, off1, brow)
            pltpu.async_copy(ft_hbm.at[ib_v], gb_v, sem_b)
            pltpu.make_async_copy(ft_hbm.at[ia_v], ga_v, sem_a).wait()

            @pl.when(k > 0)
            def _():
                pltpu.make_async_copy(oa_v, out_slice(off0 - 2 * _CHUNK), osem_a).wait()

            _combine(ga_v, wa_v, oa_v)
            pltpu.async_copy(oa_v, out_slice(off0), osem_a)

            @pl.when(k < _NPAIR - 1)
            def _():
                _compute_taps(ux_v, uy_v, ia_v, wa_v, off1 + _CHUNK, brow)
                pltpu.async_copy(ft_hbm.at[ia_v], ga_v, sem_a)

            pltpu.make_async_copy(ft_hbm.at[ib_v], gb_v, sem_b).wait()

            @pl.when(k > 0)
            def _():
                pltpu.make_async_copy(ob_v, out_slice(off1 - 2 * _CHUNK), osem_b).wait()

            _combine(gb_v, wb_v, ob_v)
            pltpu.async_copy(ob_v, out_slice(off1), osem_b)
            return 0

        lax.fori_loop(0, _NPAIR, pair, 0)

        pltpu.make_async_copy(oa_v, out_slice((_NCH - 2) * _CHUNK), osem_a).wait()
        pltpu.make_async_copy(ob_v, out_slice((_NCH - 1) * _CHUNK), osem_b).wait()

    return body(ft, ux, uy)


def kernel(f, uv):
    ft = jnp.transpose(f, (0, 2, 3, 1)).reshape(_B * _H * _W, _C)
    ft = jnp.pad(ft, ((0, 0), (0, _CP - _C)))
    ux = uv[:, :, 1].reshape(-1)
    uy = uv[:, :, 0].reshape(-1)
    out = _sc_interp(ft, ux, uy)
    return out.reshape(_B, _N, _C)
